# baseline (speedup 1.0000x reference)
import jax
import jax.numpy as jnp
from jax.experimental import pallas as pl
from jax.experimental.pallas import tpu as pltpu

_N_CHUNKS = 8


def _copy_body(in_ref, out_ref, sems):
    rows = out_ref.shape[0]
    chunk = rows // _N_CHUNKS
    for i in range(_N_CHUNKS):
        pltpu.make_async_copy(
            in_ref.at[pl.ds(i * chunk, chunk)],
            out_ref.at[pl.ds(i * chunk, chunk)],
            sems.at[i],
        ).start()
    for i in range(_N_CHUNKS):
        pltpu.make_async_copy(
            in_ref.at[pl.ds(i * chunk, chunk)],
            out_ref.at[pl.ds(i * chunk, chunk)],
            sems.at[i],
        ).wait()


def kernel(x, embeddings):
    seq_len = x.shape[1]
    hidden = embeddings.shape[1]
    return pl.pallas_call(
        _copy_body,
        out_shape=jax.ShapeDtypeStruct((seq_len, hidden), embeddings.dtype),
        in_specs=[pl.BlockSpec(memory_space=pl.ANY)],
        out_specs=pl.BlockSpec(memory_space=pl.ANY),
        scratch_shapes=[pltpu.SemaphoreType.DMA((_N_CHUNKS,))],
    )(embeddings[:seq_len])


# speedup vs baseline: 47.8532x; 47.8532x over previous
import jax
import jax.numpy as jnp
from jax.experimental import pallas as pl
from jax.experimental.pallas import tpu as pltpu

_BLOCK_ROWS = 256


def _copy_body(in_ref, out_ref):
    out_ref[...] = in_ref[...]


def kernel(x, embeddings):
    seq_len = x.shape[1]
    hidden = embeddings.shape[1]
    grid = seq_len // _BLOCK_ROWS
    return pl.pallas_call(
        _copy_body,
        out_shape=jax.ShapeDtypeStruct((seq_len, hidden), embeddings.dtype),
        grid=(grid,),
        in_specs=[pl.BlockSpec((_BLOCK_ROWS, hidden), lambda i: (i, 0))],
        out_specs=pl.BlockSpec((_BLOCK_ROWS, hidden), lambda i: (i, 0)),
    )(embeddings[:seq_len])
